# baseline (speedup 1.0000x reference)
import jax
import jax.numpy as jnp
from jax import lax
from jax.experimental import pallas as pl
from jax.experimental.pallas import tpu as pltpu

_N = 784
_BLK = 112
_G = 7
_NOUT = 100


def _body(idx_ref, I_ref, p_ref, out_ref, ip_acc):
    i = pl.program_id(0)
    ip_acc[pl.ds(i * _BLK, _BLK), :] = jnp.dot(
        I_ref[...], p_ref[...], preferred_element_type=jnp.float32)

    @pl.when(i == _G - 1)
    def _():
        v = idx_ref[...]
        flat = v[:, 0:1] * 28 + v[:, 1:2]
        cols = lax.broadcasted_iota(jnp.int32, (_NOUT, _N), 1)
        onehot = jnp.where(cols == flat, 1.0, 0.0)
        out_ref[...] = jnp.dot(onehot, ip_acc[...],
                               preferred_element_type=jnp.float32)


@jax.jit
def _run(I, p, idx2d):
    return pl.pallas_call(
        _body,
        grid=(_G,),
        in_specs=[
            pl.BlockSpec((_NOUT, 2), lambda i: (0, 0)),
            pl.BlockSpec((_BLK, _N), lambda i: (i, 0)),
            pl.BlockSpec((_N, 1), lambda i: (0, 0)),
        ],
        out_specs=pl.BlockSpec((_NOUT, 1), lambda i: (0, 0)),
        out_shape=jax.ShapeDtypeStruct((_NOUT, 1), jnp.float32),
        scratch_shapes=[pltpu.VMEM((_N, 1), jnp.float32)],
    )(idx2d, I, p.reshape(_N, 1))


def kernel(I, p, inds):
    vals = _run(I, p, inds.reshape(_NOUT, 2))
    return vals.reshape(-1, 2)
